# baseline (speedup 1.0000x reference)
import functools

import jax
import jax.numpy as jnp
from jax import lax
from jax.experimental import pallas as pl
from jax.experimental.pallas import tpu as pltpu
from jax.experimental.pallas import tpu_sc as plsc

B = 4096
F = 26
D = 64
NC = 2
NS = 16
NW = NC * NS
BW = B // NW
C = 16
NCH = BW // C
SUB = 104
NSUB = (C * F) // SUB
G = C * F


WSEG = 6248


def _fm_body(x_hbm, w0_hbm, wt_hbm, v_hbm, out_hbm,
             idx_v, idxf, vrows0, vrows1, wrows0, wrows1, tsum, out_v, w0_v,
             wtmp, wtail, w_sh,
             sem_v0, sem_v1, sem_w0, sem_w1):
    cid = lax.axis_index("c")
    sid = lax.axis_index("s")
    wid = sid * NC + cid
    xrow0 = pl.multiple_of(wid * BW, 8)

    seg = pl.multiple_of(sid * WSEG, 8)
    pltpu.sync_copy(wt_hbm.at[0, pl.ds(seg, WSEG)], wtmp)
    pltpu.sync_copy(wtmp, w_sh.at[pl.ds(seg, WSEG)])

    @pl.when(sid == 0)
    def _():
        pltpu.sync_copy(wt_hbm.at[0, pl.ds(16 * WSEG, 32)], wtail)
        pltpu.sync_copy(wtail, w_sh.at[pl.ds(16 * WSEG, 32)])

    pltpu.sync_copy(w0_hbm, w0_v)
    pltpu.sync_copy(x_hbm.at[:, pl.ds(xrow0, BW)], idx_v)

    lanes = lax.iota(jnp.int32, 16)
    row26 = lanes * F
    zeros16 = jnp.zeros((16,), jnp.int32)
    w0 = w0_v[pl.ds(0, 16)]

    def repack(bb, c):
        base = (bb * 16 + lanes) * F
        for f in range(F):
            v = idx_v[f, pl.ds(bb * 16, 16)]
            plsc.store_scatter(idxf, [base + f], v)
        return c

    lax.fori_loop(0, BW // 16, repack, 0, unroll=False)
    plsc.subcore_barrier()

    bufs = ((vrows0, wrows0, sem_v0, sem_w0),
            (vrows1, wrows1, sem_v1, sem_w1))

    def fire(g, p):
        vrows, wrows, sem_v, sem_w = bufs[p]
        for u in range(NSUB):
            off = g * G + u * SUB
            pltpu.async_copy(v_hbm.at[idxf.at[pl.ds(off, SUB)]],
                             vrows.at[pl.ds(u * SUB, SUB), :], sem_v)
            pltpu.async_copy(w_sh.at[idxf.at[pl.ds(off, SUB)]],
                             wrows.at[pl.ds(u * SUB, SUB)], sem_w)

    def drain(p):
        vrows, wrows, sem_v, sem_w = bufs[p]
        pltpu.make_async_copy(v_hbm.at[pl.ds(0, G), :], vrows, sem_v).wait()
        pltpu.make_async_copy(w_sh.at[pl.ds(0, G)], wrows, sem_w).wait()

    def compute(g, p):
        vrows, wrows, _, _ = bufs[p]

        def row_body(b, rc):
            r0 = b * F
            s_ = [vrows[r0, pl.ds(j * 16, 16)] for j in range(4)]
            q_ = [v * v for v in s_]
            for f in range(1, F):
                for j in range(4):
                    v = vrows[r0 + f, pl.ds(j * 16, 16)]
                    s_[j] = s_[j] + v
                    q_[j] = q_[j] + v * v
            t = ((s_[0] * s_[0] - q_[0]) + (s_[1] * s_[1] - q_[1])
                 + (s_[2] * s_[2] - q_[2]) + (s_[3] * s_[3] - q_[3]))
            plsc.store_scatter(tsum, [jnp.full((16,), b, jnp.int32), lanes], t)
            return rc

        lax.fori_loop(0, C, row_body, 0, unroll=False)

        first = plsc.load_gather(wrows, [row26])
        for f in range(1, F):
            first = first + plsc.load_gather(wrows, [row26 + f])

        sec = plsc.load_gather(tsum, [lanes, zeros16])
        for k in range(1, 16):
            sec = sec + plsc.load_gather(tsum, [lanes, jnp.full((16,), k, jnp.int32)])

        res = w0 + first + 0.5 * sec
        out_v[pl.ds(g * C, 16)] = res

    fire(0, 0)
    for h in range(NCH // 2):
        a, b = 2 * h, 2 * h + 1
        drain(0)
        fire(b, 1)
        compute(a, 0)
        drain(1)
        if b + 1 < NCH:
            fire(b + 1, 0)
        compute(b, 1)

    pltpu.sync_copy(out_v, out_hbm.at[pl.ds(pl.multiple_of(wid * BW, 8), BW)])


_fm = functools.partial(
    pl.kernel,
    out_type=jax.ShapeDtypeStruct((B,), jnp.float32),
    mesh=plsc.VectorSubcoreMesh(core_axis_name="c", subcore_axis_name="s",
                                num_cores=NC, num_subcores=NS),
    compiler_params=pltpu.CompilerParams(needs_layout_passes=False,
                                         use_tc_tiling_on_sc=False),
    scratch_types=[
        pltpu.VMEM((F, BW), jnp.int32),
        pltpu.VMEM((BW * F,), jnp.int32),
        pltpu.VMEM((G, D), jnp.float32),
        pltpu.VMEM((G, D), jnp.float32),
        pltpu.VMEM((G,), jnp.float32),
        pltpu.VMEM((G,), jnp.float32),
        pltpu.VMEM((16, 17), jnp.float32),
        pltpu.VMEM((BW,), jnp.float32),
        pltpu.VMEM((16,), jnp.float32),
        pltpu.VMEM((WSEG,), jnp.float32),
        pltpu.VMEM((32,), jnp.float32),
        pltpu.VMEM_SHARED((100000,), jnp.float32),
        pltpu.SemaphoreType.DMA,
        pltpu.SemaphoreType.DMA,
        pltpu.SemaphoreType.DMA,
        pltpu.SemaphoreType.DMA,
    ],
)(_fm_body)


def kernel(x, W0, W, V):
    w0b = jnp.broadcast_to(W0, (16,))
    out = _fm(x.T, w0b, W.T, V)
    return out.reshape(B, 1)


# speedup vs baseline: 1.0123x; 1.0123x over previous
import functools

import jax
import jax.numpy as jnp
from jax import lax
from jax.experimental import pallas as pl
from jax.experimental.pallas import tpu as pltpu
from jax.experimental.pallas import tpu_sc as plsc

B = 4096
F = 26
D = 64
NC = 2
NS = 16
NW = NC * NS
BW = B // NW
C = 32
NCH = BW // C
SUB = 104
NSUB = (C * F) // SUB
G = C * F


WSEG = 6248


def _fm_body(x_hbm, w0_hbm, wt_hbm, v_hbm, out_hbm,
             idx_v, idxf, vrows0, vrows1, wrows0, wrows1, tsum, out_v, w0_v,
             wtmp, wtail, w_sh,
             sem_v0, sem_v1, sem_w0, sem_w1):
    cid = lax.axis_index("c")
    sid = lax.axis_index("s")
    wid = sid * NC + cid
    xrow0 = pl.multiple_of(wid * BW, 8)

    seg = pl.multiple_of(sid * WSEG, 8)
    pltpu.sync_copy(wt_hbm.at[0, pl.ds(seg, WSEG)], wtmp)
    pltpu.sync_copy(wtmp, w_sh.at[pl.ds(seg, WSEG)])

    @pl.when(sid == 0)
    def _():
        pltpu.sync_copy(wt_hbm.at[0, pl.ds(16 * WSEG, 32)], wtail)
        pltpu.sync_copy(wtail, w_sh.at[pl.ds(16 * WSEG, 32)])

    pltpu.sync_copy(w0_hbm, w0_v)
    pltpu.sync_copy(x_hbm.at[:, pl.ds(xrow0, BW)], idx_v)

    lanes = lax.iota(jnp.int32, 16)
    row26 = lanes * F
    zeros16 = jnp.zeros((16,), jnp.int32)
    w0 = w0_v[pl.ds(0, 16)]

    def repack(bb, c):
        base = (bb * 16 + lanes) * F
        for f in range(F):
            v = idx_v[f, pl.ds(bb * 16, 16)]
            plsc.store_scatter(idxf, [base + f], v)
        return c

    lax.fori_loop(0, BW // 16, repack, 0, unroll=False)
    plsc.subcore_barrier()

    bufs = ((vrows0, wrows0, sem_v0, sem_w0),
            (vrows1, wrows1, sem_v1, sem_w1))

    def fire(g, p):
        vrows, wrows, sem_v, sem_w = bufs[p]
        for u in range(NSUB):
            off = g * G + u * SUB
            pltpu.async_copy(v_hbm.at[idxf.at[pl.ds(off, SUB)]],
                             vrows.at[pl.ds(u * SUB, SUB), :], sem_v)
            pltpu.async_copy(w_sh.at[idxf.at[pl.ds(off, SUB)]],
                             wrows.at[pl.ds(u * SUB, SUB)], sem_w)

    def drain(p):
        vrows, wrows, sem_v, sem_w = bufs[p]
        pltpu.make_async_copy(v_hbm.at[pl.ds(0, G), :], vrows, sem_v).wait()
        pltpu.make_async_copy(w_sh.at[pl.ds(0, G)], wrows, sem_w).wait()

    def compute(g, p):
        vrows, wrows, _, _ = bufs[p]

        for half in range(C // 16):
            b0 = half * 16

            def row_body(b, rc):
                r0 = b * F
                s_ = [vrows[r0, pl.ds(j * 16, 16)] for j in range(4)]
                q_ = [v * v for v in s_]
                for f in range(1, F):
                    for j in range(4):
                        v = vrows[r0 + f, pl.ds(j * 16, 16)]
                        s_[j] = s_[j] + v
                        q_[j] = q_[j] + v * v
                t = ((s_[0] * s_[0] - q_[0]) + (s_[1] * s_[1] - q_[1])
                     + (s_[2] * s_[2] - q_[2]) + (s_[3] * s_[3] - q_[3]))
                plsc.store_scatter(
                    tsum, [jnp.full((16,), b - b0, jnp.int32), lanes], t)
                return rc

            lax.fori_loop(b0, b0 + 16, row_body, 0, unroll=False)

            first = plsc.load_gather(wrows, [row26 + b0 * F])
            for f in range(1, F):
                first = first + plsc.load_gather(wrows, [row26 + (b0 * F + f)])

            sec = plsc.load_gather(tsum, [lanes, zeros16])
            for k in range(1, 16):
                sec = sec + plsc.load_gather(
                    tsum, [lanes, jnp.full((16,), k, jnp.int32)])

            res = w0 + first + 0.5 * sec
            out_v[pl.ds(g * C + b0, 16)] = res

    fire(0, 0)
    for h in range(NCH // 2):
        a, b = 2 * h, 2 * h + 1
        drain(0)
        fire(b, 1)
        compute(a, 0)
        drain(1)
        if b + 1 < NCH:
            fire(b + 1, 0)
        compute(b, 1)

    pltpu.sync_copy(out_v, out_hbm.at[pl.ds(pl.multiple_of(wid * BW, 8), BW)])


_fm = functools.partial(
    pl.kernel,
    out_type=jax.ShapeDtypeStruct((B,), jnp.float32),
    mesh=plsc.VectorSubcoreMesh(core_axis_name="c", subcore_axis_name="s",
                                num_cores=NC, num_subcores=NS),
    compiler_params=pltpu.CompilerParams(needs_layout_passes=False,
                                         use_tc_tiling_on_sc=False),
    scratch_types=[
        pltpu.VMEM((F, BW), jnp.int32),
        pltpu.VMEM((BW * F,), jnp.int32),
        pltpu.VMEM((G, D), jnp.float32),
        pltpu.VMEM((G, D), jnp.float32),
        pltpu.VMEM((G,), jnp.float32),
        pltpu.VMEM((G,), jnp.float32),
        pltpu.VMEM((16, 17), jnp.float32),
        pltpu.VMEM((BW,), jnp.float32),
        pltpu.VMEM((16,), jnp.float32),
        pltpu.VMEM((WSEG,), jnp.float32),
        pltpu.VMEM((32,), jnp.float32),
        pltpu.VMEM_SHARED((100000,), jnp.float32),
        pltpu.SemaphoreType.DMA,
        pltpu.SemaphoreType.DMA,
        pltpu.SemaphoreType.DMA,
        pltpu.SemaphoreType.DMA,
    ],
)(_fm_body)


def kernel(x, W0, W, V):
    w0b = jnp.broadcast_to(W0, (16,))
    out = _fm(x.T, w0b, W.T, V)
    return out.reshape(B, 1)


# speedup vs baseline: 1.0241x; 1.0116x over previous
import functools

import jax
import jax.numpy as jnp
from jax import lax
from jax.experimental import pallas as pl
from jax.experimental.pallas import tpu as pltpu
from jax.experimental.pallas import tpu_sc as plsc

B = 4096
F = 26
D = 64
NC = 2
NS = 16
NW = NC * NS
BW = B // NW
C = 32
NCH = BW // C
SUB = 104
NSUB = (C * F) // SUB
G = C * F


WSEG = 6248


def _fm_body(x_hbm, w0_hbm, wt_hbm, v_hbm, out_hbm,
             idx_v, idxf, vrows0, vrows1, wrows0, wrows1, tsum, out_v, w0_v,
             wtmp, wtail, w_sh,
             sem_v0, sem_v1, sem_w0, sem_w1):
    cid = lax.axis_index("c")
    sid = lax.axis_index("s")
    wid = sid * NC + cid
    xrow0 = pl.multiple_of(wid * BW, 8)

    seg = pl.multiple_of(sid * WSEG, 8)
    cp_w = pltpu.async_copy(wt_hbm.at[0, pl.ds(seg, WSEG)], wtmp, sem_w1)

    pltpu.sync_copy(w0_hbm, w0_v)
    pltpu.sync_copy(x_hbm.at[:, pl.ds(xrow0, BW)], idx_v)

    lanes = lax.iota(jnp.int32, 16)
    row26 = lanes * F
    zeros16 = jnp.zeros((16,), jnp.int32)
    w0 = w0_v[pl.ds(0, 16)]

    def repack(bb, c):
        base = (bb * 16 + lanes) * F
        for f in range(F):
            v = idx_v[f, pl.ds(bb * 16, 16)]
            plsc.store_scatter(idxf, [base + f], v)
        return c

    lax.fori_loop(0, BW // 16, repack, 0, unroll=False)

    bufs = ((vrows0, wrows0, sem_v0, sem_w0),
            (vrows1, wrows1, sem_v1, sem_w1))

    def fire_v(g, p):
        vrows, _, sem_v, _ = bufs[p]
        for u in range(NSUB):
            off = g * G + u * SUB
            pltpu.async_copy(v_hbm.at[idxf.at[pl.ds(off, SUB)]],
                             vrows.at[pl.ds(u * SUB, SUB), :], sem_v)

    def fire_w(g, p):
        _, wrows, _, sem_w = bufs[p]
        for u in range(NSUB):
            off = g * G + u * SUB
            pltpu.async_copy(w_sh.at[idxf.at[pl.ds(off, SUB)]],
                             wrows.at[pl.ds(u * SUB, SUB)], sem_w)

    def fire(g, p):
        fire_v(g, p)
        fire_w(g, p)

    fire_v(0, 0)
    cp_w.wait()
    pltpu.sync_copy(wtmp, w_sh.at[pl.ds(seg, WSEG)])

    @pl.when(sid == 0)
    def _():
        pltpu.sync_copy(wt_hbm.at[0, pl.ds(16 * WSEG, 32)], wtail)
        pltpu.sync_copy(wtail, w_sh.at[pl.ds(16 * WSEG, 32)])

    plsc.subcore_barrier()
    fire_w(0, 0)

    def drain(p):
        vrows, wrows, sem_v, sem_w = bufs[p]
        pltpu.make_async_copy(v_hbm.at[pl.ds(0, G), :], vrows, sem_v).wait()
        pltpu.make_async_copy(w_sh.at[pl.ds(0, G)], wrows, sem_w).wait()

    def compute(g, p):
        vrows, wrows, _, _ = bufs[p]

        for half in range(C // 16):
            b0 = half * 16

            def row_body(b, rc):
                r0 = b * F
                s_ = [vrows[r0, pl.ds(j * 16, 16)] for j in range(4)]
                q_ = [v * v for v in s_]
                for f in range(1, F):
                    for j in range(4):
                        v = vrows[r0 + f, pl.ds(j * 16, 16)]
                        s_[j] = s_[j] + v
                        q_[j] = q_[j] + v * v
                t = ((s_[0] * s_[0] - q_[0]) + (s_[1] * s_[1] - q_[1])
                     + (s_[2] * s_[2] - q_[2]) + (s_[3] * s_[3] - q_[3]))
                plsc.store_scatter(
                    tsum, [jnp.full((16,), b - b0, jnp.int32), lanes], t)
                return rc

            lax.fori_loop(b0, b0 + 16, row_body, 0, unroll=False)

            first = plsc.load_gather(wrows, [row26 + b0 * F])
            for f in range(1, F):
                first = first + plsc.load_gather(wrows, [row26 + (b0 * F + f)])

            sec = plsc.load_gather(tsum, [lanes, zeros16])
            for k in range(1, 16):
                sec = sec + plsc.load_gather(
                    tsum, [lanes, jnp.full((16,), k, jnp.int32)])

            res = w0 + first + 0.5 * sec
            out_v[pl.ds(g * C + b0, 16)] = res

    for h in range(NCH // 2):
        a, b = 2 * h, 2 * h + 1
        drain(0)
        fire(b, 1)
        compute(a, 0)
        drain(1)
        if b + 1 < NCH:
            fire(b + 1, 0)
        compute(b, 1)

    pltpu.sync_copy(out_v, out_hbm.at[pl.ds(pl.multiple_of(wid * BW, 8), BW)])


_fm = functools.partial(
    pl.kernel,
    out_type=jax.ShapeDtypeStruct((B,), jnp.float32),
    mesh=plsc.VectorSubcoreMesh(core_axis_name="c", subcore_axis_name="s",
                                num_cores=NC, num_subcores=NS),
    compiler_params=pltpu.CompilerParams(needs_layout_passes=False,
                                         use_tc_tiling_on_sc=False),
    scratch_types=[
        pltpu.VMEM((F, BW), jnp.int32),
        pltpu.VMEM((BW * F,), jnp.int32),
        pltpu.VMEM((G, D), jnp.float32),
        pltpu.VMEM((G, D), jnp.float32),
        pltpu.VMEM((G,), jnp.float32),
        pltpu.VMEM((G,), jnp.float32),
        pltpu.VMEM((16, 17), jnp.float32),
        pltpu.VMEM((BW,), jnp.float32),
        pltpu.VMEM((16,), jnp.float32),
        pltpu.VMEM((WSEG,), jnp.float32),
        pltpu.VMEM((32,), jnp.float32),
        pltpu.VMEM_SHARED((100000,), jnp.float32),
        pltpu.SemaphoreType.DMA,
        pltpu.SemaphoreType.DMA,
        pltpu.SemaphoreType.DMA,
        pltpu.SemaphoreType.DMA,
    ],
)(_fm_body)


def kernel(x, W0, W, V):
    w0b = jnp.broadcast_to(W0, (16,))
    out = _fm(x.T, w0b, W.T, V)
    return out.reshape(B, 1)
